# baseline (speedup 1.0000x reference)
import functools

import jax
import jax.numpy as jnp
from jax import lax
from jax.experimental import pallas as pl
from jax.experimental.pallas import tpu as pltpu
from jax.experimental.pallas import tpu_sc as plsc

N = 50000
E = 800000
HID = 64
HALF = 32

NTILES = 16
LANES = 128
KG = 2
E_PAD = 802816
EROWS = E_PAD // LANES
RPT = EROWS // NTILES
NCHUNK = RPT // KG
R_PAD = 50000
RNODE = R_PAD // NTILES
DROWS = 125

BN = 2000


def _mlp_in(beta, degree, W_in, b_in):
    def body(b_ref, d_ref, w_ref, bias_ref, h0_ref, h1_ref):
        b = b_ref[...]
        d = d_ref[...]
        w = w_ref[...]
        bias = bias_ref[...]
        h = b * w[0:1, :] + (b * b) * w[1:2, :] + d * w[2:3, :] + bias
        h = jnp.maximum(h, 0.0)
        h0_ref[...] = h[:, :HALF]
        h1_ref[...] = h[:, HALF:]

    return pl.pallas_call(
        body,
        grid=(N // BN,),
        in_specs=[
            pl.BlockSpec((BN, 1), lambda i: (i, 0)),
            pl.BlockSpec((BN, 1), lambda i: (i, 0)),
            pl.BlockSpec((3, HID), lambda i: (0, 0)),
            pl.BlockSpec((1, HID), lambda i: (0, 0)),
        ],
        out_specs=[
            pl.BlockSpec((BN, HALF), lambda i: (i, 0)),
            pl.BlockSpec((BN, HALF), lambda i: (i, 0)),
        ],
        out_shape=[
            jax.ShapeDtypeStruct((N, HALF), jnp.float32),
            jax.ShapeDtypeStruct((N, HALF), jnp.float32),
        ],
    )(beta, degree, W_in, b_in.reshape(1, HID))


def _mlp_out(ah0, ah1, a2h0, a2h1, w1a, w1b, w2a, w2b, W_out, b_out):
    def body(a0_ref, a1_ref, b0_ref, b1_ref, w1a_ref, w1b_ref, w2a_ref,
             w2b_ref, wo_ref, bo_ref, g_ref):
        h2 = (
            jnp.dot(a0_ref[...], w1a_ref[...], preferred_element_type=jnp.float32,
                      precision=lax.Precision.HIGHEST)
            + jnp.dot(a1_ref[...], w1b_ref[...], preferred_element_type=jnp.float32,
                      precision=lax.Precision.HIGHEST)
            + jnp.dot(b0_ref[...], w2a_ref[...], preferred_element_type=jnp.float32,
                      precision=lax.Precision.HIGHEST)
            + jnp.dot(b1_ref[...], w2b_ref[...], preferred_element_type=jnp.float32,
                      precision=lax.Precision.HIGHEST)
        )
        h2 = jnp.maximum(h2, 0.0)
        z = jnp.dot(h2, wo_ref[...], preferred_element_type=jnp.float32,
                      precision=lax.Precision.HIGHEST) + bo_ref[...]
        g_ref[...] = jnp.maximum(z, 0.0) + jnp.log(1.0 + jnp.exp(-jnp.abs(z)))

    full = lambda shape: pl.BlockSpec(shape, lambda i: (0, 0))
    return pl.pallas_call(
        body,
        grid=(N // BN,),
        in_specs=[
            pl.BlockSpec((BN, HALF), lambda i: (i, 0)),
            pl.BlockSpec((BN, HALF), lambda i: (i, 0)),
            pl.BlockSpec((BN, HALF), lambda i: (i, 0)),
            pl.BlockSpec((BN, HALF), lambda i: (i, 0)),
            full((HALF, HID)),
            full((HALF, HID)),
            full((HALF, HID)),
            full((HALF, HID)),
            full((HID, 1)),
            full((1, 1)),
        ],
        out_specs=pl.BlockSpec((BN, 1), lambda i: (i, 0)),
        out_shape=jax.ShapeDtypeStruct((N, 1), jnp.float32),
    )(ah0, ah1, a2h0, a2h1, w1a, w1b, w2a, w2b, W_out, b_out.reshape(1, 1))


def _spmm2(h0, h1, edata):
    mesh = plsc.VectorSubcoreMesh(core_axis_name="c", subcore_axis_name="s")
    out_f = jax.ShapeDtypeStruct((R_PAD, HALF), jnp.float32)
    out_b = jax.ShapeDtypeStruct((R_PAD, HALF // 2), jnp.int32)

    @functools.partial(
        pl.kernel,
        mesh=mesh,
        out_type=[out_f, out_f, out_f, out_f, out_b, out_b],
        compiler_params=pltpu.CompilerParams(
            use_tc_tiling_on_sc=False, needs_layout_passes=False),
        scratch_types=[
            pltpu.VMEM((KG, 3, LANES), jnp.int32),
            pltpu.VMEM((KG, 3, LANES), jnp.int32),
            pltpu.VMEM((KG, LANES, HALF // 2), jnp.int32),
            pltpu.VMEM((KG, LANES, HALF // 2), jnp.int32),
            pltpu.VMEM((KG, LANES, HALF), jnp.float32),
            pltpu.VMEM((DROWS, HALF), jnp.float32),
            pltpu.VMEM((DROWS, HALF // 2), jnp.int32),
            pltpu.VMEM_SHARED((R_PAD, HALF), jnp.float32),
            pltpu.SemaphoreType.DMA,
            pltpu.SemaphoreType.DMA,
        ],
    )
    def sc(h0r, h1r, edr, ah0r, ah1r, a2h0r, a2h1r, ab0r, ab1r,
           eb0, eb1, rb0, rb1, rf, stage, stbf, acc, sem0, sem1):
        c = lax.axis_index("c")
        s = lax.axis_index("s")
        zvec = jnp.zeros((16,), jnp.float32)

        def memset_stage():
            def zb(r, carry):
                stage[r, pl.ds(0, 16)] = zvec
                stage[r, pl.ds(16, 16)] = zvec
                return carry
            lax.fori_loop(0, DROWS, zb, 0)

        def zero_acc():
            for q in range(RNODE // DROWS):
                pltpu.sync_copy(stage,
                                acc.at[pl.ds(s * RNODE + q * DROWS, DROWS)])

        memset_stage()
        zero_acc()
        plsc.subcore_barrier()

        bufs = ((eb0, rb0, sem0), (eb1, rb1, sem1))

        def edge_pass(table):
            def load_chunk(ci, eb):
                base = s * RPT + ci * KG
                pltpu.sync_copy(edr.at[pl.ds(base, KG)], eb)

            def fire(eb, rb, sem):
                for j in range(KG):
                    pltpu.async_copy(table.at[eb.at[j, 0]], rb.at[j], sem)

            def wait_g(eb, rb, sem):
                for j in range(KG):
                    pltpu.make_async_copy(
                        table.at[eb.at[j, 0]], rb.at[j], sem).wait()

            def consume(eb, rb):
                emask = jnp.full((16,), 0x0FFFE000, jnp.int32)
                ebias = jnp.full((16,), 0x38000000, jnp.int32)
                for j in range(KG):
                    def body(b, carry2):
                        w16 = plsc.bitcast(eb[j, 2, pl.ds(b * 16, 16)],
                                           jnp.float32)
                        for u in range(16):
                            e = b * 16 + u
                            wv = w16[u]
                            v = rb[j, e, pl.ds(0, 16)]
                            ev = plsc.bitcast(
                                ((v << 13) & emask) + ebias, jnp.float32)
                            od = plsc.bitcast(
                                ((v >> 3) & emask) + ebias, jnp.float32)
                            rf[j, e, pl.ds(0, 16)] = ev * wv
                            rf[j, e, pl.ds(16, 16)] = od * wv
                        return carry2
                    lax.fori_loop(0, LANES // 16, body, 0)
                for j in range(KG):
                    pltpu.sync_copy(rf.at[j], acc.at[eb.at[j, 1]], add=True)

            load_chunk(0, eb0)
            fire(eb0, rb0, sem0)

            def pair(p, carry):
                for half in range(2):
                    ci = p * 2 + half
                    eb, rb, sem = bufs[half]
                    ebn, rbn, semn = bufs[1 - half]

                    @pl.when(ci + 1 < NCHUNK)
                    def _():
                        load_chunk(ci + 1, ebn)
                        fire(ebn, rbn, semn)

                    wait_g(eb, rb, sem)
                    consume(eb, rb)
                return carry

            lax.fori_loop(0, NCHUNK // 2, pair, 0)

        def dump(out_ref, bf_ref):
            for q in range(RNODE // DROWS):
                r0 = s * RNODE + q * DROWS
                pltpu.sync_copy(acc.at[pl.ds(r0, DROWS)], stage)
                pltpu.sync_copy(stage, out_ref.at[pl.ds(r0, DROWS)])
                if bf_ref is not None:
                    ebias = jnp.full((16,), 0x38000000, jnp.int32)
                    zero = jnp.zeros((16,), jnp.int32)
                    def repack(r, carry):
                        ev = plsc.bitcast(stage[r, pl.ds(0, 16)], jnp.int32)
                        od = plsc.bitcast(stage[r, pl.ds(16, 16)], jnp.int32)
                        lo = jnp.maximum(ev - ebias, zero) >> 13
                        hi = jnp.maximum(od - ebias, zero) >> 13
                        stbf[r, pl.ds(0, 16)] = lo | (hi << 16)
                        return carry
                    lax.fori_loop(0, DROWS, repack, 0)
                    pltpu.sync_copy(stbf, bf_ref.at[pl.ds(r0, DROWS)])
            memset_stage()
            zero_acc()

        @pl.when(c == 0)
        def _():
            edge_pass(h0r)

        @pl.when(c == 1)
        def _():
            edge_pass(h1r)

        plsc.subcore_barrier()

        @pl.when(c == 0)
        def _():
            dump(ah0r, ab0r)

        @pl.when(c == 1)
        def _():
            dump(ah1r, ab1r)

        plsc.subcore_barrier()

        @pl.when(c == 0)
        def _():
            edge_pass(ab0r)

        @pl.when(c == 1)
        def _():
            edge_pass(ab1r)

        plsc.subcore_barrier()

        @pl.when(c == 0)
        def _():
            dump(a2h0r, None)

        @pl.when(c == 1)
        def _():
            dump(a2h1r, None)

    return sc(h0, h1, edata)[:4]


def kernel(beta, degree, edge_index, edge_weight, W_in, b_in, W_mp1, W_mp2,
           W_out, b_out):
    pad = E_PAD - E
    srcm = jnp.pad(edge_index[0], (0, pad)).reshape(EROWS, LANES)
    dstm = jnp.pad(edge_index[1], (0, pad)).reshape(EROWS, LANES)
    wm = lax.bitcast_convert_type(
        jnp.pad(edge_weight, (0, pad)).reshape(EROWS, LANES), jnp.int32)
    edata = jnp.stack([srcm, dstm, wm], axis=1)

    h0, h1 = _mlp_in(beta, degree, W_in, b_in)
    h0 = lax.bitcast_convert_type(
        h0.astype(jnp.float16).reshape(N, HALF // 2, 2), jnp.int32)
    h1 = lax.bitcast_convert_type(
        h1.astype(jnp.float16).reshape(N, HALF // 2, 2), jnp.int32)
    ah0, ah1, a2h0, a2h1 = _spmm2(h0, h1, edata)
    perm = jnp.arange(HALF).reshape(HALF // 2, 2).T.reshape(-1)
    return _mlp_out(
        ah0, ah1, a2h0, a2h1,
        W_mp1[:HALF][perm], W_mp1[HALF:][perm],
        W_mp2[:HALF][perm], W_mp2[HALF:][perm],
        W_out, b_out,
    )


# speedup vs baseline: 1.7266x; 1.7266x over previous
import functools

import jax
import jax.numpy as jnp
from jax import lax
from jax.experimental import pallas as pl
from jax.experimental.pallas import tpu as pltpu
from jax.experimental.pallas import tpu_sc as plsc

N = 50000
E = 800000
HID = 64
HALF = 32

NTILES = 16
LANES = 128
KG = 2
E_PAD = 802816
EROWS = E_PAD // LANES
RPT = EROWS // NTILES
NCHUNK = RPT // KG
R_PAD = 50000
RNODE = R_PAD // NTILES
DROWS = 125

BN = 2000


def _mlp_in(beta, degree, W_in, b_in):
    def body(b_ref, d_ref, w_ref, bias_ref, h0_ref, h1_ref):
        b = b_ref[...]
        d = d_ref[...]
        w = w_ref[...]
        bias = bias_ref[...]
        h = b * w[0:1, :] + (b * b) * w[1:2, :] + d * w[2:3, :] + bias
        h = jnp.maximum(h, 0.0)
        h0_ref[...] = h[:, :HALF]
        h1_ref[...] = h[:, HALF:]

    return pl.pallas_call(
        body,
        grid=(N // BN,),
        in_specs=[
            pl.BlockSpec((BN, 1), lambda i: (i, 0)),
            pl.BlockSpec((BN, 1), lambda i: (i, 0)),
            pl.BlockSpec((3, HID), lambda i: (0, 0)),
            pl.BlockSpec((1, HID), lambda i: (0, 0)),
        ],
        out_specs=[
            pl.BlockSpec((BN, HALF), lambda i: (i, 0)),
            pl.BlockSpec((BN, HALF), lambda i: (i, 0)),
        ],
        out_shape=[
            jax.ShapeDtypeStruct((N, HALF), jnp.float32),
            jax.ShapeDtypeStruct((N, HALF), jnp.float32),
        ],
    )(beta, degree, W_in, b_in.reshape(1, HID))


def _mlp_out(ah0, ah1, a2h0, a2h1, w1a, w1b, w2a, w2b, W_out, b_out):
    def body(a0_ref, a1_ref, b0_ref, b1_ref, w1a_ref, w1b_ref, w2a_ref,
             w2b_ref, wo_ref, bo_ref, g_ref):
        h2 = (
            jnp.dot(a0_ref[...], w1a_ref[...], preferred_element_type=jnp.float32,
                      precision=lax.Precision.HIGHEST)
            + jnp.dot(a1_ref[...], w1b_ref[...], preferred_element_type=jnp.float32,
                      precision=lax.Precision.HIGHEST)
            + jnp.dot(b0_ref[...], w2a_ref[...], preferred_element_type=jnp.float32,
                      precision=lax.Precision.HIGHEST)
            + jnp.dot(b1_ref[...], w2b_ref[...], preferred_element_type=jnp.float32,
                      precision=lax.Precision.HIGHEST)
        )
        h2 = jnp.maximum(h2, 0.0)
        z = jnp.dot(h2, wo_ref[...], preferred_element_type=jnp.float32,
                      precision=lax.Precision.HIGHEST) + bo_ref[...]
        g_ref[...] = jnp.maximum(z, 0.0) + jnp.log(1.0 + jnp.exp(-jnp.abs(z)))

    full = lambda shape: pl.BlockSpec(shape, lambda i: (0, 0))
    return pl.pallas_call(
        body,
        grid=(N // BN,),
        in_specs=[
            pl.BlockSpec((BN, HALF), lambda i: (i, 0)),
            pl.BlockSpec((BN, HALF), lambda i: (i, 0)),
            pl.BlockSpec((BN, HALF), lambda i: (i, 0)),
            pl.BlockSpec((BN, HALF), lambda i: (i, 0)),
            full((HALF, HID)),
            full((HALF, HID)),
            full((HALF, HID)),
            full((HALF, HID)),
            full((HID, 1)),
            full((1, 1)),
        ],
        out_specs=pl.BlockSpec((BN, 1), lambda i: (i, 0)),
        out_shape=jax.ShapeDtypeStruct((N, 1), jnp.float32),
    )(ah0, ah1, a2h0, a2h1, w1a, w1b, w2a, w2b, W_out, b_out.reshape(1, 1))


def _spmm2(h0, h1, edata):
    mesh = plsc.VectorSubcoreMesh(core_axis_name="c", subcore_axis_name="s")
    out = jax.ShapeDtypeStruct((R_PAD, HALF), jnp.float32)

    @functools.partial(
        pl.kernel,
        mesh=mesh,
        out_type=[out, out, out, out],
        compiler_params=pltpu.CompilerParams(
            use_tc_tiling_on_sc=False, needs_layout_passes=False),
        scratch_types=[
            pltpu.VMEM((KG, 3, LANES), jnp.int32),
            pltpu.VMEM((KG, 3, LANES), jnp.int32),
            pltpu.VMEM((KG, LANES, HALF), jnp.float32),
            pltpu.VMEM((KG, LANES, HALF), jnp.float32),
            pltpu.VMEM((DROWS, HALF), jnp.float32),
            pltpu.VMEM((DROWS, HALF), jnp.float32),
            pltpu.VMEM_SHARED((R_PAD, HALF), jnp.float32),
            pltpu.SemaphoreType.DMA,
            pltpu.SemaphoreType.DMA,
            pltpu.SemaphoreType.DMA,
            pltpu.SemaphoreType.DMA,
        ],
    )
    def sc(h0r, h1r, edr, ah0r, ah1r, a2h0r, a2h1r,
           eb0, eb1, rw0, rw1, stage, zbuf, acc, sem0, sem1, ssem0, ssem1):
        c = lax.axis_index("c")
        s = lax.axis_index("s")
        zvec = jnp.zeros((16,), jnp.float32)

        def zb_init(r, carry):
            zbuf[r, pl.ds(0, 16)] = zvec
            zbuf[r, pl.ds(16, 16)] = zvec
            return carry

        lax.fori_loop(0, DROWS, zb_init, 0)

        for q in range(RNODE // DROWS):
            pltpu.sync_copy(zbuf, acc.at[pl.ds(s * RNODE + q * DROWS, DROWS)])
        plsc.subcore_barrier()

        bufs = ((eb0, rw0, sem0, ssem0), (eb1, rw1, sem1, ssem1))

        def edge_pass(table):
            def load_chunk(ci, eb):
                base = s * RPT + ci * KG
                pltpu.sync_copy(edr.at[pl.ds(base, KG)], eb)

            def fire(eb, rw, sem):
                for j in range(KG):
                    pltpu.async_copy(table.at[eb.at[j, 0]], rw.at[j], sem)

            def wait_g(eb, rw, sem):
                for j in range(KG):
                    pltpu.make_async_copy(
                        table.at[eb.at[j, 0]], rw.at[j], sem).wait()

            def scale(eb, rw):
                for j in range(KG):
                    def body(b, carry2):
                        w16 = plsc.bitcast(eb[j, 2, pl.ds(b * 16, 16)],
                                           jnp.float32)
                        for u in range(16):
                            e = b * 16 + u
                            wv = w16[u]
                            rw[j, e, pl.ds(0, 16)] = rw[j, e, pl.ds(0, 16)] * wv
                            rw[j, e, pl.ds(16, 16)] = rw[j, e, pl.ds(16, 16)] * wv
                        return carry2
                    lax.fori_loop(0, LANES // 16, body, 0)

            def fire_s(eb, rw, ssem):
                for j in range(KG):
                    pltpu.async_copy(rw.at[j], acc.at[eb.at[j, 1]], ssem,
                                     add=True)

            def wait_s(eb, rw, ssem):
                for j in range(KG):
                    pltpu.make_async_copy(rw.at[j], acc.at[eb.at[j, 1]],
                                          ssem).wait()

            load_chunk(0, eb0)
            fire(eb0, rw0, sem0)

            def pair(p, carry):
                for half in range(2):
                    ci = p * 2 + half
                    eb, rw, sem, ssem = bufs[half]
                    ebn, rwn, semn, ssemn = bufs[1 - half]

                    @pl.when(ci + 1 < NCHUNK)
                    def _():
                        @pl.when(ci >= 1)
                        def _():
                            wait_s(ebn, rwn, ssemn)
                        load_chunk(ci + 1, ebn)
                        fire(ebn, rwn, semn)

                    wait_g(eb, rw, sem)
                    scale(eb, rw)
                    fire_s(eb, rw, ssem)
                return carry

            lax.fori_loop(0, NCHUNK // 2, pair, 0)
            wait_s(eb0, rw0, ssem0)
            wait_s(eb1, rw1, ssem1)

        def dump(out_ref):
            for q in range(RNODE // DROWS):
                r0 = s * RNODE + q * DROWS
                pltpu.sync_copy(acc.at[pl.ds(r0, DROWS)], stage)
                pltpu.sync_copy(stage, out_ref.at[pl.ds(r0, DROWS)])
                pltpu.sync_copy(zbuf, acc.at[pl.ds(r0, DROWS)])

        @pl.when(c == 0)
        def _():
            edge_pass(h0r)

        @pl.when(c == 1)
        def _():
            edge_pass(h1r)

        plsc.subcore_barrier()

        @pl.when(c == 0)
        def _():
            dump(ah0r)

        @pl.when(c == 1)
        def _():
            dump(ah1r)

        plsc.subcore_barrier()

        @pl.when(c == 0)
        def _():
            edge_pass(ah0r)

        @pl.when(c == 1)
        def _():
            edge_pass(ah1r)

        plsc.subcore_barrier()

        @pl.when(c == 0)
        def _():
            dump(a2h0r)

        @pl.when(c == 1)
        def _():
            dump(a2h1r)

    return sc(h0, h1, edata)


def kernel(beta, degree, edge_index, edge_weight, W_in, b_in, W_mp1, W_mp2,
           W_out, b_out):
    pad = E_PAD - E
    srcm = jnp.pad(edge_index[0], (0, pad)).reshape(EROWS, LANES)
    dstm = jnp.pad(edge_index[1], (0, pad)).reshape(EROWS, LANES)
    wm = lax.bitcast_convert_type(
        jnp.pad(edge_weight, (0, pad)).reshape(EROWS, LANES), jnp.int32)
    edata = jnp.stack([srcm, dstm, wm], axis=1)

    h0, h1 = _mlp_in(beta, degree, W_in, b_in)
    ah0, ah1, a2h0, a2h1 = _spmm2(h0, h1, edata)
    return _mlp_out(
        ah0, ah1, a2h0, a2h1,
        W_mp1[:HALF], W_mp1[HALF:], W_mp2[:HALF], W_mp2[HALF:],
        W_out, b_out,
    )


# speedup vs baseline: 1.9133x; 1.1081x over previous
import functools

import jax
import jax.numpy as jnp
from jax import lax
from jax.experimental import pallas as pl
from jax.experimental.pallas import tpu as pltpu
from jax.experimental.pallas import tpu_sc as plsc

N = 50000
E = 800000
HID = 64
HALF = 32

NTILES = 16
LANES = 128
KG = 2
E_PAD = 802816
EROWS = E_PAD // LANES
RPT = EROWS // NTILES
NCHUNK = RPT // KG
R_PAD = 50000
RNODE = R_PAD // NTILES
DROWS = 125

BN = 2000


def _mlp_in(beta, degree, W_in, b_in):
    def body(b_ref, d_ref, w_ref, bias_ref, h0_ref, h1_ref):
        b = b_ref[...]
        d = d_ref[...]
        w = w_ref[...]
        bias = bias_ref[...]
        h = b * w[0:1, :] + (b * b) * w[1:2, :] + d * w[2:3, :] + bias
        h = jnp.maximum(h, 0.0)
        h0_ref[...] = h[:, :HALF]
        h1_ref[...] = h[:, HALF:]

    return pl.pallas_call(
        body,
        grid=(N // BN,),
        in_specs=[
            pl.BlockSpec((BN, 1), lambda i: (i, 0)),
            pl.BlockSpec((BN, 1), lambda i: (i, 0)),
            pl.BlockSpec((3, HID), lambda i: (0, 0)),
            pl.BlockSpec((1, HID), lambda i: (0, 0)),
        ],
        out_specs=[
            pl.BlockSpec((BN, HALF), lambda i: (i, 0)),
            pl.BlockSpec((BN, HALF), lambda i: (i, 0)),
        ],
        out_shape=[
            jax.ShapeDtypeStruct((N, HALF), jnp.float32),
            jax.ShapeDtypeStruct((N, HALF), jnp.float32),
        ],
    )(beta, degree, W_in, b_in.reshape(1, HID))


def _mlp_out(ah0, ah1, a2h0, a2h1, w1a, w1b, w2a, w2b, W_out, b_out):
    def body(a0_ref, a1_ref, b0_ref, b1_ref, w1a_ref, w1b_ref, w2a_ref,
             w2b_ref, wo_ref, bo_ref, g_ref):
        h2 = (
            jnp.dot(a0_ref[...], w1a_ref[...], preferred_element_type=jnp.float32)
            + jnp.dot(a1_ref[...], w1b_ref[...], preferred_element_type=jnp.float32)
            + jnp.dot(b0_ref[...], w2a_ref[...], preferred_element_type=jnp.float32)
            + jnp.dot(b1_ref[...], w2b_ref[...], preferred_element_type=jnp.float32)
        )
        h2 = jnp.maximum(h2, 0.0)
        z = jnp.dot(h2, wo_ref[...], preferred_element_type=jnp.float32) + bo_ref[...]
        g_ref[...] = jnp.maximum(z, 0.0) + jnp.log(1.0 + jnp.exp(-jnp.abs(z)))

    full = lambda shape: pl.BlockSpec(shape, lambda i: (0, 0))
    return pl.pallas_call(
        body,
        grid=(N // BN,),
        in_specs=[
            pl.BlockSpec((BN, HALF), lambda i: (i, 0)),
            pl.BlockSpec((BN, HALF), lambda i: (i, 0)),
            pl.BlockSpec((BN, HALF), lambda i: (i, 0)),
            pl.BlockSpec((BN, HALF), lambda i: (i, 0)),
            full((HALF, HID)),
            full((HALF, HID)),
            full((HALF, HID)),
            full((HALF, HID)),
            full((HID, 1)),
            full((1, 1)),
        ],
        out_specs=pl.BlockSpec((BN, 1), lambda i: (i, 0)),
        out_shape=jax.ShapeDtypeStruct((N, 1), jnp.float32),
    )(ah0, ah1, a2h0, a2h1, w1a, w1b, w2a, w2b, W_out, b_out.reshape(1, 1))


def _spmm2(h0, h1, edata):
    mesh = plsc.VectorSubcoreMesh(core_axis_name="c", subcore_axis_name="s")
    out = jax.ShapeDtypeStruct((R_PAD, HALF), jnp.float32)

    @functools.partial(
        pl.kernel,
        mesh=mesh,
        out_type=[out, out, out, out],
        compiler_params=pltpu.CompilerParams(
            use_tc_tiling_on_sc=False, needs_layout_passes=False),
        scratch_types=[
            pltpu.VMEM((KG, 3, LANES), jnp.int32),
            pltpu.VMEM((KG, 3, LANES), jnp.int32),
            pltpu.VMEM((KG, LANES, HALF), jnp.float32),
            pltpu.VMEM((KG, LANES, HALF), jnp.float32),
            pltpu.VMEM((DROWS, HALF), jnp.float32),
            pltpu.VMEM((DROWS, HALF), jnp.float32),
            pltpu.VMEM_SHARED((R_PAD, HALF), jnp.float32),
            pltpu.SemaphoreType.DMA,
            pltpu.SemaphoreType.DMA,
            pltpu.SemaphoreType.DMA,
            pltpu.SemaphoreType.DMA,
        ],
    )
    def sc(h0r, h1r, edr, ah0r, ah1r, a2h0r, a2h1r,
           eb0, eb1, rw0, rw1, stage, zbuf, acc, sem0, sem1, ssem0, ssem1):
        c = lax.axis_index("c")
        s = lax.axis_index("s")
        zvec = jnp.zeros((16,), jnp.float32)

        def zb_init(r, carry):
            zbuf[r, pl.ds(0, 16)] = zvec
            zbuf[r, pl.ds(16, 16)] = zvec
            return carry

        lax.fori_loop(0, DROWS, zb_init, 0)

        for q in range(RNODE // DROWS):
            pltpu.sync_copy(zbuf, acc.at[pl.ds(s * RNODE + q * DROWS, DROWS)])
        plsc.subcore_barrier()

        bufs = ((eb0, rw0, sem0, ssem0), (eb1, rw1, sem1, ssem1))

        def edge_pass(table):
            def load_chunk(ci, eb):
                base = s * RPT + ci * KG
                pltpu.sync_copy(edr.at[pl.ds(base, KG)], eb)

            def fire(eb, rw, sem):
                for j in range(KG):
                    pltpu.async_copy(table.at[eb.at[j, 0]], rw.at[j], sem)

            def wait_g(eb, rw, sem):
                for j in range(KG):
                    pltpu.make_async_copy(
                        table.at[eb.at[j, 0]], rw.at[j], sem).wait()

            def scale(eb, rw):
                for j in range(KG):
                    def body(b, carry2):
                        w16 = plsc.bitcast(eb[j, 2, pl.ds(b * 16, 16)],
                                           jnp.float32)
                        for u in range(16):
                            e = b * 16 + u
                            wv = w16[u]
                            rw[j, e, pl.ds(0, 16)] = rw[j, e, pl.ds(0, 16)] * wv
                            rw[j, e, pl.ds(16, 16)] = rw[j, e, pl.ds(16, 16)] * wv
                        return carry2
                    lax.fori_loop(0, LANES // 16, body, 0)

            def fire_s(eb, rw, ssem):
                for j in range(KG):
                    pltpu.async_copy(rw.at[j], acc.at[eb.at[j, 1]], ssem,
                                     add=True)

            def wait_s(eb, rw, ssem):
                for j in range(KG):
                    pltpu.make_async_copy(rw.at[j], acc.at[eb.at[j, 1]],
                                          ssem).wait()

            load_chunk(0, eb0)
            fire(eb0, rw0, sem0)

            def pair(p, carry):
                for half in range(2):
                    ci = p * 2 + half
                    eb, rw, sem, ssem = bufs[half]
                    ebn, rwn, semn, ssemn = bufs[1 - half]

                    @pl.when(ci + 1 < NCHUNK)
                    def _():
                        @pl.when(ci >= 1)
                        def _():
                            wait_s(ebn, rwn, ssemn)
                        load_chunk(ci + 1, ebn)
                        fire(ebn, rwn, semn)

                    wait_g(eb, rw, sem)
                    scale(eb, rw)
                    fire_s(eb, rw, ssem)
                return carry

            lax.fori_loop(0, NCHUNK // 2, pair, 0)
            wait_s(eb0, rw0, ssem0)
            wait_s(eb1, rw1, ssem1)

        def dump(out_ref):
            for q in range(RNODE // DROWS):
                r0 = s * RNODE + q * DROWS
                pltpu.sync_copy(acc.at[pl.ds(r0, DROWS)], stage)
                pltpu.sync_copy(stage, out_ref.at[pl.ds(r0, DROWS)])
                pltpu.sync_copy(zbuf, acc.at[pl.ds(r0, DROWS)])

        @pl.when(c == 0)
        def _():
            edge_pass(h0r)

        @pl.when(c == 1)
        def _():
            edge_pass(h1r)

        plsc.subcore_barrier()

        @pl.when(c == 0)
        def _():
            dump(ah0r)

        @pl.when(c == 1)
        def _():
            dump(ah1r)

        plsc.subcore_barrier()

        @pl.when(c == 0)
        def _():
            edge_pass(ah0r)

        @pl.when(c == 1)
        def _():
            edge_pass(ah1r)

        plsc.subcore_barrier()

        @pl.when(c == 0)
        def _():
            dump(a2h0r)

        @pl.when(c == 1)
        def _():
            dump(a2h1r)

    return sc(h0, h1, edata)


def kernel(beta, degree, edge_index, edge_weight, W_in, b_in, W_mp1, W_mp2,
           W_out, b_out):
    pad = E_PAD - E
    srcm = jnp.pad(edge_index[0], (0, pad)).reshape(EROWS, LANES)
    dstm = jnp.pad(edge_index[1], (0, pad)).reshape(EROWS, LANES)
    wm = lax.bitcast_convert_type(
        jnp.pad(edge_weight, (0, pad)).reshape(EROWS, LANES), jnp.int32)
    edata = jnp.stack([srcm, dstm, wm], axis=1)

    h0, h1 = _mlp_in(beta, degree, W_in, b_in)
    ah0, ah1, a2h0, a2h1 = _spmm2(h0, h1, edata)
    return _mlp_out(
        ah0, ah1, a2h0, a2h1,
        W_mp1[:HALF], W_mp1[HALF:], W_mp2[:HALF], W_mp2[HALF:],
        W_out, b_out,
    )
